# Initial kernel scaffold; baseline (speedup 1.0000x reference)
import jax
import jax.numpy as jnp
from jax.experimental import pallas as pl


def kernel(data, segment_ids):
    raise NotImplementedError("write your pallas kernel here")


# speedup vs baseline: 4.1992x; 4.1992x over previous
import functools

import jax
import jax.numpy as jnp
from jax import lax
from jax.experimental import pallas as pl
from jax.experimental.pallas import tpu as pltpu
from jax.experimental.pallas import tpu_sc as plsc

N = 320000
D = 128
S = 10000

IDROW = 128
IDROWS = N // IDROW
IDROWS_PAD = 2504
NW = 16
UNITS = IDROWS // 8
BIGW = 8
DBROWS = 256
SEG_PER_TILE = 624
SEG_TAIL = S - 16 * SEG_PER_TILE

_mesh = plsc.VectorSubcoreMesh(core_axis_name="c", subcore_axis_name="s",
                               num_cores=1)


@functools.partial(
    pl.kernel,
    out_type=jax.ShapeDtypeStruct((S, D), jnp.float32),
    mesh=_mesh,
    scratch_types=[
        pltpu.VMEM((DBROWS, D), jnp.float32),
        pltpu.VMEM((8, IDROW), jnp.int32),
        pltpu.VMEM_SHARED((S, D), jnp.float32),
    ],
)
def _seg_sum_sc(data_hbm, ids_hbm, zeros_hbm, out_hbm, dbuf, ibuf, acc_sh):
    s = lax.axis_index("s")
    w = s

    pltpu.sync_copy(zeros_hbm, dbuf)
    base = pl.multiple_of(s * SEG_PER_TILE, 8)
    pltpu.sync_copy(dbuf, acc_sh.at[pl.ds(base, DBROWS)])
    pltpu.sync_copy(dbuf, acc_sh.at[pl.ds(base + DBROWS, DBROWS)])
    pltpu.sync_copy(dbuf.at[pl.ds(0, SEG_PER_TILE - 2 * DBROWS)],
                    acc_sh.at[pl.ds(base + 2 * DBROWS,
                                    SEG_PER_TILE - 2 * DBROWS)])

    @pl.when(s == 0)
    def _zero_tail():
        pltpu.sync_copy(dbuf.at[pl.ds(0, SEG_TAIL)],
                        acc_sh.at[pl.ds(16 * SEG_PER_TILE, SEG_TAIL)])

    plsc.subcore_barrier()

    start_unit = w * 20 - jnp.maximum(w - BIGW, 0)
    n_units = 20 - (w >= BIGW).astype(jnp.int32)

    def unit_body(u, carry):
        unit = start_unit + u
        idrow0 = pl.multiple_of(unit * 8, 8)
        pltpu.sync_copy(ids_hbm.at[pl.ds(idrow0, 8)], ibuf)
        for j in range(4):
            row0 = pl.multiple_of(unit * 1024 + j * DBROWS, 8)
            pltpu.sync_copy(data_hbm.at[pl.ds(row0, DBROWS)], dbuf)
            for h in range(2):
                pltpu.sync_copy(dbuf.at[pl.ds(h * IDROW, IDROW)],
                                acc_sh.at[ibuf.at[2 * j + h]], add=True)
        return carry

    lax.fori_loop(0, n_units, unit_body, 0)

    @pl.when(w == NW - 1)
    def _tail():
        pltpu.sync_copy(ids_hbm.at[pl.ds(UNITS * 8, 8)], ibuf)
        for j in range(2):
            row0 = pl.multiple_of(UNITS * 1024 + j * DBROWS, 8)
            pltpu.sync_copy(data_hbm.at[pl.ds(row0, DBROWS)], dbuf)
            for h in range(2):
                pltpu.sync_copy(dbuf.at[pl.ds(h * IDROW, IDROW)],
                                acc_sh.at[ibuf.at[2 * j + h]], add=True)

    plsc.subcore_barrier()

    pltpu.sync_copy(acc_sh.at[pl.ds(base, DBROWS)], dbuf)
    pltpu.sync_copy(dbuf, out_hbm.at[pl.ds(base, DBROWS)])
    pltpu.sync_copy(acc_sh.at[pl.ds(base + DBROWS, DBROWS)], dbuf)
    pltpu.sync_copy(dbuf, out_hbm.at[pl.ds(base + DBROWS, DBROWS)])
    pltpu.sync_copy(acc_sh.at[pl.ds(base + 2 * DBROWS,
                                    SEG_PER_TILE - 2 * DBROWS)],
                    dbuf.at[pl.ds(0, SEG_PER_TILE - 2 * DBROWS)])
    pltpu.sync_copy(dbuf.at[pl.ds(0, SEG_PER_TILE - 2 * DBROWS)],
                    out_hbm.at[pl.ds(base + 2 * DBROWS,
                                     SEG_PER_TILE - 2 * DBROWS)])

    @pl.when(s == 0)
    def _write_tail():
        pltpu.sync_copy(acc_sh.at[pl.ds(16 * SEG_PER_TILE, SEG_TAIL)],
                        dbuf.at[pl.ds(0, SEG_TAIL)])
        pltpu.sync_copy(dbuf.at[pl.ds(0, SEG_TAIL)],
                        out_hbm.at[pl.ds(16 * SEG_PER_TILE, SEG_TAIL)])


def kernel(data, segment_ids):
    ids2d = segment_ids.astype(jnp.int32).reshape(IDROWS, IDROW)
    ids2d = jnp.pad(ids2d, ((0, IDROWS_PAD - IDROWS), (0, 0)))
    zeros = jnp.zeros((DBROWS, D), jnp.float32)
    return _seg_sum_sc(data, ids2d, zeros)


# speedup vs baseline: 5.2440x; 1.2488x over previous
import functools

import jax
import jax.numpy as jnp
from jax import lax
from jax.experimental import pallas as pl
from jax.experimental.pallas import tpu as pltpu
from jax.experimental.pallas import tpu_sc as plsc

N = 320000
D = 128
S = 10000

IDROW = 128
IDROWS = N // IDROW
IDROWS_PAD = 2504
NW = 16
UNITS = IDROWS // 8
BIGW = 8
SEG_PER_TILE = 624
SEG_TAIL = S - 16 * SEG_PER_TILE

_mesh = plsc.VectorSubcoreMesh(core_axis_name="c", subcore_axis_name="s",
                               num_cores=1)


@functools.partial(
    pl.kernel,
    out_type=jax.ShapeDtypeStruct((S, D), jnp.float32),
    mesh=_mesh,
    scratch_types=[
        pltpu.VMEM((IDROW, D), jnp.float32),
        pltpu.VMEM((IDROW, D), jnp.float32),
        pltpu.VMEM((8, IDROW), jnp.int32),
        pltpu.VMEM_SHARED((S, D), jnp.float32),
        pltpu.SemaphoreType.DMA,
        pltpu.SemaphoreType.DMA,
    ],
)
def _seg_sum_sc(data_hbm, ids_hbm, zeros_hbm, out_hbm, dbuf0, dbuf1, ibuf,
                acc_sh, sem0, sem1):
    s = lax.axis_index("s")
    w = s
    bufs = (dbuf0, dbuf1)
    sems = (sem0, sem1)

    pltpu.sync_copy(zeros_hbm, dbuf0)
    base = pl.multiple_of(s * SEG_PER_TILE, 8)
    for k in range(4):
        pltpu.sync_copy(dbuf0, acc_sh.at[pl.ds(base + k * IDROW, IDROW)])
    pltpu.sync_copy(dbuf0.at[pl.ds(0, SEG_PER_TILE - 4 * IDROW)],
                    acc_sh.at[pl.ds(base + 4 * IDROW,
                                    SEG_PER_TILE - 4 * IDROW)])

    @pl.when(s == 0)
    def _zero_tail():
        pltpu.sync_copy(dbuf0.at[pl.ds(0, SEG_TAIL)],
                        acc_sh.at[pl.ds(16 * SEG_PER_TILE, SEG_TAIL)])

    plsc.subcore_barrier()

    start_unit = w * 20 - jnp.maximum(w - BIGW, 0)
    n_units = 20 - (w >= BIGW).astype(jnp.int32)
    block0 = start_unit * 8

    def rowof(b):
        return pl.multiple_of(jnp.minimum(b * IDROW, N - IDROW), 8)

    pltpu.sync_copy(ids_hbm.at[pl.ds(pl.multiple_of(block0, 8), 8)], ibuf)
    pltpu.async_copy(data_hbm.at[pl.ds(rowof(block0), IDROW)], dbuf0, sem0)

    def unit_body(u, carry):
        for j in range(8):
            b = block0 + u * 8 + j
            buf, sem = bufs[j % 2], sems[j % 2]
            obuf, osem = bufs[1 - j % 2], sems[1 - j % 2]
            pltpu.make_async_copy(data_hbm.at[pl.ds(rowof(b), IDROW)],
                                  buf, sem).wait()
            pltpu.async_copy(data_hbm.at[pl.ds(rowof(b + 1), IDROW)],
                             obuf, osem)
            pltpu.sync_copy(buf, acc_sh.at[ibuf.at[j]], add=True)
        nxt = pl.multiple_of(
            jnp.minimum((start_unit + u + 1) * 8, IDROWS_PAD - 8), 8)
        pltpu.sync_copy(ids_hbm.at[pl.ds(nxt, 8)], ibuf)
        return carry

    lax.fori_loop(0, n_units, unit_body, 0)

    pltpu.make_async_copy(
        data_hbm.at[pl.ds(rowof(block0 + n_units * 8), IDROW)],
        dbuf0, sem0).wait()

    @pl.when(w == NW - 1)
    def _tail():
        pltpu.sync_copy(ids_hbm.at[pl.ds(UNITS * 8, 8)], ibuf)
        for j in range(4):
            row0 = pl.multiple_of((UNITS * 8 + j) * IDROW, 8)
            pltpu.sync_copy(data_hbm.at[pl.ds(row0, IDROW)], dbuf0)
            pltpu.sync_copy(dbuf0, acc_sh.at[ibuf.at[j]], add=True)

    plsc.subcore_barrier()

    for k in range(4):
        pltpu.sync_copy(acc_sh.at[pl.ds(base + k * IDROW, IDROW)], dbuf0)
        pltpu.sync_copy(dbuf0, out_hbm.at[pl.ds(base + k * IDROW, IDROW)])
    pltpu.sync_copy(acc_sh.at[pl.ds(base + 4 * IDROW,
                                    SEG_PER_TILE - 4 * IDROW)],
                    dbuf0.at[pl.ds(0, SEG_PER_TILE - 4 * IDROW)])
    pltpu.sync_copy(dbuf0.at[pl.ds(0, SEG_PER_TILE - 4 * IDROW)],
                    out_hbm.at[pl.ds(base + 4 * IDROW,
                                     SEG_PER_TILE - 4 * IDROW)])

    @pl.when(s == 0)
    def _write_tail():
        pltpu.sync_copy(acc_sh.at[pl.ds(16 * SEG_PER_TILE, SEG_TAIL)],
                        dbuf0.at[pl.ds(0, SEG_TAIL)])
        pltpu.sync_copy(dbuf0.at[pl.ds(0, SEG_TAIL)],
                        out_hbm.at[pl.ds(16 * SEG_PER_TILE, SEG_TAIL)])


def kernel(data, segment_ids):
    ids2d = segment_ids.astype(jnp.int32).reshape(IDROWS, IDROW)
    ids2d = jnp.pad(ids2d, ((0, IDROWS_PAD - IDROWS), (0, 0)))
    zeros = jnp.zeros((IDROW, D), jnp.float32)
    return _seg_sum_sc(data, ids2d, zeros)


# speedup vs baseline: 5.3049x; 1.0116x over previous
import functools

import jax
import jax.numpy as jnp
from jax import lax
from jax.experimental import pallas as pl
from jax.experimental.pallas import tpu as pltpu
from jax.experimental.pallas import tpu_sc as plsc

N = 320000
D = 128
S = 10000

IDROW = 128
IDROWS = N // IDROW
IDROWS_PAD = 2504
NW = 16
UNITS = IDROWS // 8
BIGW = 8
SEG_PER_TILE = 624
SEG_TAIL = S - 16 * SEG_PER_TILE

_mesh = plsc.VectorSubcoreMesh(core_axis_name="c", subcore_axis_name="s",
                               num_cores=1)


@functools.partial(
    pl.kernel,
    out_type=jax.ShapeDtypeStruct((S, D), jnp.float32),
    mesh=_mesh,
    scratch_types=[
        pltpu.VMEM((IDROW, D), jnp.float32),
        pltpu.VMEM((IDROW, D), jnp.float32),
        pltpu.VMEM((2, 8, IDROW), jnp.int32),
        pltpu.VMEM_SHARED((S, D), jnp.float32),
        pltpu.SemaphoreType.DMA,
        pltpu.SemaphoreType.DMA,
        pltpu.SemaphoreType.DMA,
        pltpu.SemaphoreType.DMA,
    ],
)
def _seg_sum_sc(data_hbm, ids_hbm, zeros_hbm, out_hbm, dbuf0, dbuf1, ibufs,
                acc_sh, sl0, sl1, ss0, ss1):
    s = lax.axis_index("s")
    w = s
    bufs = (dbuf0, dbuf1)
    lsems = (sl0, sl1)
    ssems = (ss0, ss1)

    pltpu.sync_copy(zeros_hbm, dbuf0)
    base = pl.multiple_of(s * SEG_PER_TILE, 8)
    for k in range(4):
        pltpu.sync_copy(dbuf0, acc_sh.at[pl.ds(base + k * IDROW, IDROW)])
    pltpu.sync_copy(dbuf0.at[pl.ds(0, SEG_PER_TILE - 4 * IDROW)],
                    acc_sh.at[pl.ds(base + 4 * IDROW,
                                    SEG_PER_TILE - 4 * IDROW)])

    @pl.when(s == 0)
    def _zero_tail():
        pltpu.sync_copy(dbuf0.at[pl.ds(0, SEG_TAIL)],
                        acc_sh.at[pl.ds(16 * SEG_PER_TILE, SEG_TAIL)])

    plsc.subcore_barrier()

    start_unit = w * 20 - jnp.maximum(w - BIGW, 0)
    n_units = 20 - (w >= BIGW).astype(jnp.int32)
    block0 = start_unit * 8

    def rowof(b):
        return pl.multiple_of(jnp.minimum(b * IDROW, N - IDROW), 8)

    pltpu.sync_copy(ids_hbm.at[pl.ds(pl.multiple_of(block0, 8), 8)],
                    ibufs.at[0])
    pltpu.async_copy(data_hbm.at[pl.ds(rowof(block0), IDROW)], dbuf0, sl0)

    def unit_body(u, carry):
        up = lax.rem(u, 2)
        ub = block0 + u * 8
        for j in range(8):
            m = j % 2
            buf, lsem, ssem = bufs[m], lsems[m], ssems[m]
            obuf, olsem, ossem = bufs[1 - m], lsems[1 - m], ssems[1 - m]
            pltpu.make_async_copy(data_hbm.at[pl.ds(rowof(ub + j), IDROW)],
                                  buf, lsem).wait()
            pltpu.async_copy(buf, acc_sh.at[ibufs.at[up, j]], ssem, add=True)
            if j == 0:
                @pl.when(u > 0)
                def _():
                    pltpu.make_async_copy(
                        dbuf1, acc_sh.at[ibufs.at[1 - up, 7]], ss1).wait()
            else:
                pltpu.make_async_copy(
                    obuf, acc_sh.at[ibufs.at[up, j - 1]], ossem).wait()
            pltpu.async_copy(data_hbm.at[pl.ds(rowof(ub + j + 1), IDROW)],
                             obuf, olsem)
        nxt = pl.multiple_of(
            jnp.minimum((start_unit + u + 1) * 8, IDROWS_PAD - 8), 8)
        pltpu.sync_copy(ids_hbm.at[pl.ds(nxt, 8)], ibufs.at[1 - up])
        return carry

    lax.fori_loop(0, n_units, unit_body, 0)

    up_last = lax.rem(n_units - 1, 2)
    pltpu.make_async_copy(dbuf1, acc_sh.at[ibufs.at[up_last, 7]], ss1).wait()
    pltpu.make_async_copy(
        data_hbm.at[pl.ds(rowof(block0 + n_units * 8), IDROW)],
        dbuf0, sl0).wait()

    @pl.when(w == NW - 1)
    def _tail():
        pltpu.sync_copy(ids_hbm.at[pl.ds(UNITS * 8, 8)], ibufs.at[0])
        for j in range(4):
            row0 = pl.multiple_of((UNITS * 8 + j) * IDROW, 8)
            pltpu.sync_copy(data_hbm.at[pl.ds(row0, IDROW)], dbuf0)
            pltpu.sync_copy(dbuf0, acc_sh.at[ibufs.at[0, j]], add=True)

    plsc.subcore_barrier()

    for k in range(4):
        pltpu.sync_copy(acc_sh.at[pl.ds(base + k * IDROW, IDROW)], dbuf0)
        pltpu.sync_copy(dbuf0, out_hbm.at[pl.ds(base + k * IDROW, IDROW)])
    pltpu.sync_copy(acc_sh.at[pl.ds(base + 4 * IDROW,
                                    SEG_PER_TILE - 4 * IDROW)],
                    dbuf0.at[pl.ds(0, SEG_PER_TILE - 4 * IDROW)])
    pltpu.sync_copy(dbuf0.at[pl.ds(0, SEG_PER_TILE - 4 * IDROW)],
                    out_hbm.at[pl.ds(base + 4 * IDROW,
                                     SEG_PER_TILE - 4 * IDROW)])

    @pl.when(s == 0)
    def _write_tail():
        pltpu.sync_copy(acc_sh.at[pl.ds(16 * SEG_PER_TILE, SEG_TAIL)],
                        dbuf0.at[pl.ds(0, SEG_TAIL)])
        pltpu.sync_copy(dbuf0.at[pl.ds(0, SEG_TAIL)],
                        out_hbm.at[pl.ds(16 * SEG_PER_TILE, SEG_TAIL)])


def kernel(data, segment_ids):
    ids2d = segment_ids.astype(jnp.int32).reshape(IDROWS, IDROW)
    ids2d = jnp.pad(ids2d, ((0, IDROWS_PAD - IDROWS), (0, 0)))
    zeros = jnp.zeros((IDROW, D), jnp.float32)
    return _seg_sum_sc(data, ids2d, zeros)


# speedup vs baseline: 8.4522x; 1.5933x over previous
import functools

import jax
import jax.numpy as jnp
from jax import lax
from jax.experimental import pallas as pl
from jax.experimental.pallas import tpu as pltpu
from jax.experimental.pallas import tpu_sc as plsc

N = 320000
D = 128
S = 10000

IDROW = 128
IDROWS = N // IDROW
IDROWS_PAD = 2504
NW = 32
UNITS = IDROWS // 8
BIGW = 24
SEG_PER_TILE = 624
SEG_TAIL = S - 16 * SEG_PER_TILE

_mesh = plsc.VectorSubcoreMesh(core_axis_name="c", subcore_axis_name="s")


@functools.partial(
    pl.kernel,
    out_type=jax.ShapeDtypeStruct((2, S, D), jnp.float32),
    mesh=_mesh,
    scratch_types=[
        pltpu.VMEM((IDROW, D), jnp.float32),
        pltpu.VMEM((IDROW, D), jnp.float32),
        pltpu.VMEM((2, 8, IDROW), jnp.int32),
        pltpu.VMEM_SHARED((S, D), jnp.float32),
        pltpu.SemaphoreType.DMA,
        pltpu.SemaphoreType.DMA,
        pltpu.SemaphoreType.DMA,
        pltpu.SemaphoreType.DMA,
    ],
)
def _seg_sum_sc(data_hbm, ids_hbm, zeros_hbm, out_hbm, dbuf0, dbuf1, ibufs,
                acc_sh, sl0, sl1, ss0, ss1):
    c = lax.axis_index("c")
    s = lax.axis_index("s")
    w = c * 16 + s
    bufs = (dbuf0, dbuf1)
    lsems = (sl0, sl1)
    ssems = (ss0, ss1)

    pltpu.sync_copy(zeros_hbm, dbuf0)
    base = pl.multiple_of(s * SEG_PER_TILE, 8)
    for k in range(4):
        pltpu.sync_copy(dbuf0, acc_sh.at[pl.ds(base + k * IDROW, IDROW)])
    pltpu.sync_copy(dbuf0.at[pl.ds(0, SEG_PER_TILE - 4 * IDROW)],
                    acc_sh.at[pl.ds(base + 4 * IDROW,
                                    SEG_PER_TILE - 4 * IDROW)])

    @pl.when(s == 0)
    def _zero_tail():
        pltpu.sync_copy(dbuf0.at[pl.ds(0, SEG_TAIL)],
                        acc_sh.at[pl.ds(16 * SEG_PER_TILE, SEG_TAIL)])

    plsc.subcore_barrier()

    start_unit = w * 10 - jnp.maximum(w - BIGW, 0)
    n_units = 10 - (w >= BIGW).astype(jnp.int32)
    block0 = start_unit * 8

    def rowof(b):
        return pl.multiple_of(jnp.minimum(b * IDROW, N - IDROW), 8)

    pltpu.sync_copy(ids_hbm.at[pl.ds(pl.multiple_of(block0, 8), 8)],
                    ibufs.at[0])
    pltpu.async_copy(data_hbm.at[pl.ds(rowof(block0), IDROW)], dbuf0, sl0)

    def unit_body(u, carry):
        up = lax.rem(u, 2)
        ub = block0 + u * 8
        for j in range(8):
            m = j % 2
            buf, lsem, ssem = bufs[m], lsems[m], ssems[m]
            obuf, olsem, ossem = bufs[1 - m], lsems[1 - m], ssems[1 - m]
            pltpu.make_async_copy(data_hbm.at[pl.ds(rowof(ub + j), IDROW)],
                                  buf, lsem).wait()
            pltpu.async_copy(buf, acc_sh.at[ibufs.at[up, j]], ssem, add=True)
            if j == 0:
                @pl.when(u > 0)
                def _():
                    pltpu.make_async_copy(
                        dbuf1, acc_sh.at[ibufs.at[1 - up, 7]], ss1).wait()
            else:
                pltpu.make_async_copy(
                    obuf, acc_sh.at[ibufs.at[up, j - 1]], ossem).wait()
            pltpu.async_copy(data_hbm.at[pl.ds(rowof(ub + j + 1), IDROW)],
                             obuf, olsem)
        nxt = pl.multiple_of(
            jnp.minimum((start_unit + u + 1) * 8, IDROWS_PAD - 8), 8)
        pltpu.sync_copy(ids_hbm.at[pl.ds(nxt, 8)], ibufs.at[1 - up])
        return carry

    lax.fori_loop(0, n_units, unit_body, 0)

    up_last = lax.rem(n_units - 1, 2)
    pltpu.make_async_copy(dbuf1, acc_sh.at[ibufs.at[up_last, 7]], ss1).wait()
    pltpu.make_async_copy(
        data_hbm.at[pl.ds(rowof(block0 + n_units * 8), IDROW)],
        dbuf0, sl0).wait()

    @pl.when(w == NW - 1)
    def _tail():
        pltpu.sync_copy(ids_hbm.at[pl.ds(UNITS * 8, 8)], ibufs.at[0])
        for j in range(4):
            row0 = pl.multiple_of((UNITS * 8 + j) * IDROW, 8)
            pltpu.sync_copy(data_hbm.at[pl.ds(row0, IDROW)], dbuf0)
            pltpu.sync_copy(dbuf0, acc_sh.at[ibufs.at[0, j]], add=True)

    plsc.subcore_barrier()

    for k in range(4):
        pltpu.sync_copy(acc_sh.at[pl.ds(base + k * IDROW, IDROW)], dbuf0)
        pltpu.sync_copy(dbuf0, out_hbm.at[c, pl.ds(base + k * IDROW, IDROW)])
    pltpu.sync_copy(acc_sh.at[pl.ds(base + 4 * IDROW,
                                    SEG_PER_TILE - 4 * IDROW)],
                    dbuf0.at[pl.ds(0, SEG_PER_TILE - 4 * IDROW)])
    pltpu.sync_copy(dbuf0.at[pl.ds(0, SEG_PER_TILE - 4 * IDROW)],
                    out_hbm.at[c, pl.ds(base + 4 * IDROW,
                                        SEG_PER_TILE - 4 * IDROW)])

    @pl.when(s == 0)
    def _write_tail():
        pltpu.sync_copy(acc_sh.at[pl.ds(16 * SEG_PER_TILE, SEG_TAIL)],
                        dbuf0.at[pl.ds(0, SEG_TAIL)])
        pltpu.sync_copy(dbuf0.at[pl.ds(0, SEG_TAIL)],
                        out_hbm.at[c, pl.ds(16 * SEG_PER_TILE, SEG_TAIL)])


def _combine_body(p_ref, o_ref):
    o_ref[...] = p_ref[0] + p_ref[1]


def _combine(partials):
    return pl.pallas_call(
        _combine_body,
        grid=(10,),
        in_specs=[pl.BlockSpec((2, S // 10, D), lambda i: (0, i, 0))],
        out_specs=pl.BlockSpec((S // 10, D), lambda i: (i, 0)),
        out_shape=jax.ShapeDtypeStruct((S, D), jnp.float32),
    )(partials)


def kernel(data, segment_ids):
    ids2d = segment_ids.astype(jnp.int32).reshape(IDROWS, IDROW)
    ids2d = jnp.pad(ids2d, ((0, IDROWS_PAD - IDROWS), (0, 0)))
    zeros = jnp.zeros((IDROW, D), jnp.float32)
    partials = _seg_sum_sc(data, ids2d, zeros)
    return _combine(partials)


# speedup vs baseline: 8.6014x; 1.0177x over previous
import functools

import jax
import jax.numpy as jnp
from jax import lax
from jax.experimental import pallas as pl
from jax.experimental.pallas import tpu as pltpu
from jax.experimental.pallas import tpu_sc as plsc

N = 320000
D = 128
S = 10000

IDROW = 128
IDROWS = N // IDROW
IDROWS_PAD = 2504
NW = 32
UNITS = IDROWS // 8
BIGW = 24
SEG_PER_TILE = 624
SEG_TAIL = S - 16 * SEG_PER_TILE

_mesh = plsc.VectorSubcoreMesh(core_axis_name="c", subcore_axis_name="s")


@functools.partial(
    pl.kernel,
    out_type=jax.ShapeDtypeStruct((2, S, D), jnp.float32),
    mesh=_mesh,
    scratch_types=[
        pltpu.VMEM((IDROW, D), jnp.float32),
        pltpu.VMEM((IDROW, D), jnp.float32),
        pltpu.VMEM((2, 8, IDROW), jnp.int32),
        pltpu.VMEM_SHARED((S, D), jnp.float32),
        pltpu.SemaphoreType.DMA,
        pltpu.SemaphoreType.DMA,
        pltpu.SemaphoreType.DMA,
        pltpu.SemaphoreType.DMA,
    ],
)
def _seg_sum_sc(data_hbm, ids_hbm, zeros_hbm, out_hbm, dbuf0, dbuf1, ibufs,
                acc_sh, sl0, sl1, ss0, ss1):
    c = lax.axis_index("c")
    s = lax.axis_index("s")
    w = c * 16 + s
    bufs = (dbuf0, dbuf1)
    lsems = (sl0, sl1)
    ssems = (ss0, ss1)

    pltpu.sync_copy(zeros_hbm, dbuf0)
    base = pl.multiple_of(s * SEG_PER_TILE, 8)
    ztail = SEG_PER_TILE - 4 * IDROW
    for k in range(4):
        pltpu.async_copy(dbuf0, acc_sh.at[pl.ds(base + k * IDROW, IDROW)],
                         ss0)
    pltpu.async_copy(dbuf0.at[pl.ds(0, ztail)],
                     acc_sh.at[pl.ds(base + 4 * IDROW, ztail)], ss1)

    @pl.when(s == 0)
    def _zero_tail():
        pltpu.sync_copy(dbuf0.at[pl.ds(0, SEG_TAIL)],
                        acc_sh.at[pl.ds(16 * SEG_PER_TILE, SEG_TAIL)])

    for k in range(4):
        pltpu.make_async_copy(
            dbuf0, acc_sh.at[pl.ds(base + k * IDROW, IDROW)], ss0).wait()
    pltpu.make_async_copy(dbuf0.at[pl.ds(0, ztail)],
                          acc_sh.at[pl.ds(base + 4 * IDROW, ztail)],
                          ss1).wait()

    plsc.subcore_barrier()

    start_unit = w * 10 - jnp.maximum(w - BIGW, 0)
    n_units = 10 - (w >= BIGW).astype(jnp.int32)
    block0 = start_unit * 8

    def rowof(b):
        return pl.multiple_of(jnp.minimum(b * IDROW, N - IDROW), 8)

    pltpu.sync_copy(ids_hbm.at[pl.ds(pl.multiple_of(block0, 8), 8)],
                    ibufs.at[0])
    pltpu.async_copy(data_hbm.at[pl.ds(rowof(block0), IDROW)], dbuf0, sl0)

    def unit_body(u, carry):
        up = lax.rem(u, 2)
        ub = block0 + u * 8
        for j in range(8):
            m = j % 2
            buf, lsem, ssem = bufs[m], lsems[m], ssems[m]
            obuf, olsem, ossem = bufs[1 - m], lsems[1 - m], ssems[1 - m]
            pltpu.make_async_copy(data_hbm.at[pl.ds(rowof(ub + j), IDROW)],
                                  buf, lsem).wait()
            pltpu.async_copy(buf, acc_sh.at[ibufs.at[up, j]], ssem, add=True)
            if j == 0:
                @pl.when(u > 0)
                def _():
                    pltpu.make_async_copy(
                        dbuf1, acc_sh.at[ibufs.at[1 - up, 7]], ss1).wait()
            else:
                pltpu.make_async_copy(
                    obuf, acc_sh.at[ibufs.at[up, j - 1]], ossem).wait()
            pltpu.async_copy(data_hbm.at[pl.ds(rowof(ub + j + 1), IDROW)],
                             obuf, olsem)
        nxt = pl.multiple_of(
            jnp.minimum((start_unit + u + 1) * 8, IDROWS_PAD - 8), 8)
        pltpu.sync_copy(ids_hbm.at[pl.ds(nxt, 8)], ibufs.at[1 - up])
        return carry

    lax.fori_loop(0, n_units, unit_body, 0)

    up_last = lax.rem(n_units - 1, 2)
    pltpu.make_async_copy(dbuf1, acc_sh.at[ibufs.at[up_last, 7]], ss1).wait()
    pltpu.make_async_copy(
        data_hbm.at[pl.ds(rowof(block0 + n_units * 8), IDROW)],
        dbuf0, sl0).wait()

    @pl.when(w == NW - 1)
    def _tail():
        pltpu.sync_copy(ids_hbm.at[pl.ds(UNITS * 8, 8)], ibufs.at[0])
        for j in range(4):
            row0 = pl.multiple_of((UNITS * 8 + j) * IDROW, 8)
            pltpu.sync_copy(data_hbm.at[pl.ds(row0, IDROW)], dbuf0)
            pltpu.sync_copy(dbuf0, acc_sh.at[ibufs.at[0, j]], add=True)

    plsc.subcore_barrier()

    rtail = SEG_PER_TILE - 4 * IDROW

    def _racc(k):
        rows = IDROW if k < 4 else rtail
        return acc_sh.at[pl.ds(base + k * IDROW, rows)]

    def _rbuf(k):
        buf = bufs[k % 2]
        return buf if k < 4 else buf.at[pl.ds(0, rtail)]

    def _rout(k):
        rows = IDROW if k < 4 else rtail
        return out_hbm.at[c, pl.ds(base + k * IDROW, rows)]

    pltpu.async_copy(_racc(0), _rbuf(0), lsems[0])
    for k in range(5):
        pltpu.make_async_copy(_racc(k), _rbuf(k), lsems[k % 2]).wait()
        if k >= 1:
            pltpu.make_async_copy(_rbuf(k - 1), _rout(k - 1),
                                  ssems[(k - 1) % 2]).wait()
        if k < 4:
            pltpu.async_copy(_racc(k + 1), _rbuf(k + 1), lsems[(k + 1) % 2])
        pltpu.async_copy(_rbuf(k), _rout(k), ssems[k % 2])
    pltpu.make_async_copy(_rbuf(4), _rout(4), ssems[0]).wait()

    @pl.when(s == 0)
    def _write_tail():
        pltpu.sync_copy(acc_sh.at[pl.ds(16 * SEG_PER_TILE, SEG_TAIL)],
                        dbuf0.at[pl.ds(0, SEG_TAIL)])
        pltpu.sync_copy(dbuf0.at[pl.ds(0, SEG_TAIL)],
                        out_hbm.at[c, pl.ds(16 * SEG_PER_TILE, SEG_TAIL)])


def _combine_body(p_ref, o_ref):
    o_ref[...] = p_ref[0] + p_ref[1]


def _combine(partials):
    return pl.pallas_call(
        _combine_body,
        grid=(10,),
        in_specs=[pl.BlockSpec((2, S // 10, D), lambda i: (0, i, 0))],
        out_specs=pl.BlockSpec((S // 10, D), lambda i: (i, 0)),
        out_shape=jax.ShapeDtypeStruct((S, D), jnp.float32),
    )(partials)


def kernel(data, segment_ids):
    ids2d = segment_ids.astype(jnp.int32).reshape(IDROWS, IDROW)
    ids2d = jnp.pad(ids2d, ((0, IDROWS_PAD - IDROWS), (0, 0)))
    zeros = jnp.zeros((IDROW, D), jnp.float32)
    partials = _seg_sum_sc(data, ids2d, zeros)
    return _combine(partials)


# speedup vs baseline: 8.8262x; 1.0261x over previous
import functools

import jax
import jax.numpy as jnp
from jax import lax
from jax.experimental import pallas as pl
from jax.experimental.pallas import tpu as pltpu
from jax.experimental.pallas import tpu_sc as plsc

N = 320000
D = 128
S = 10000

IDROW = 128
IDROWS = N // IDROW
IDROWS_PAD = 2504
NW = 32
UNITS = IDROWS // 8
BIGW = 24
SEG_PER_TILE = 624
SEG_TAIL = S - 16 * SEG_PER_TILE

_mesh = plsc.VectorSubcoreMesh(core_axis_name="c", subcore_axis_name="s")


@functools.partial(
    pl.kernel,
    out_type=jax.ShapeDtypeStruct((2, S, D), jnp.float32),
    mesh=_mesh,
    scratch_types=[
        pltpu.VMEM((IDROW, D), jnp.float32),
        pltpu.VMEM((IDROW, D), jnp.float32),
        pltpu.VMEM((2, 8, IDROW), jnp.int32),
        pltpu.VMEM_SHARED((S, D), jnp.float32),
        pltpu.SemaphoreType.DMA,
        pltpu.SemaphoreType.DMA,
        pltpu.SemaphoreType.DMA,
        pltpu.SemaphoreType.DMA,
    ],
)
def _seg_sum_sc(data_hbm, ids_hbm, zeros_hbm, out_hbm, dbuf0, dbuf1, ibufs,
                acc_sh, sl0, sl1, ss0, ss1):
    c = lax.axis_index("c")
    s = lax.axis_index("s")
    w = c * 16 + s
    bufs = (dbuf0, dbuf1)
    lsems = (sl0, sl1)
    ssems = (ss0, ss1)

    start_unit = w * 10 - jnp.maximum(w - BIGW, 0)
    n_units = 10 - (w >= BIGW).astype(jnp.int32)
    block0 = start_unit * 8

    def rowof(b):
        return pl.multiple_of(jnp.minimum(b * IDROW, N - IDROW), 8)

    pltpu.async_copy(data_hbm.at[pl.ds(rowof(block0), IDROW)], dbuf0, sl0)
    pltpu.async_copy(ids_hbm.at[pl.ds(pl.multiple_of(block0, 8), 8)],
                     ibufs.at[0], sl1)

    pltpu.sync_copy(zeros_hbm, dbuf1)
    base = pl.multiple_of(s * SEG_PER_TILE, 8)
    ztail = SEG_PER_TILE - 4 * IDROW
    for k in range(4):
        pltpu.async_copy(dbuf1, acc_sh.at[pl.ds(base + k * IDROW, IDROW)],
                         ss0)
    pltpu.async_copy(dbuf1.at[pl.ds(0, ztail)],
                     acc_sh.at[pl.ds(base + 4 * IDROW, ztail)], ss1)

    @pl.when(s == 0)
    def _zero_tail():
        pltpu.sync_copy(dbuf1.at[pl.ds(0, SEG_TAIL)],
                        acc_sh.at[pl.ds(16 * SEG_PER_TILE, SEG_TAIL)])

    for k in range(4):
        pltpu.make_async_copy(
            dbuf1, acc_sh.at[pl.ds(base + k * IDROW, IDROW)], ss0).wait()
    pltpu.make_async_copy(dbuf1.at[pl.ds(0, ztail)],
                          acc_sh.at[pl.ds(base + 4 * IDROW, ztail)],
                          ss1).wait()
    pltpu.make_async_copy(ids_hbm.at[pl.ds(pl.multiple_of(block0, 8), 8)],
                          ibufs.at[0], sl1).wait()

    plsc.subcore_barrier()

    def unit_body(u, carry):
        up = lax.rem(u, 2)
        ub = block0 + u * 8
        for j in range(8):
            m = j % 2
            buf, lsem, ssem = bufs[m], lsems[m], ssems[m]
            obuf, olsem, ossem = bufs[1 - m], lsems[1 - m], ssems[1 - m]
            pltpu.make_async_copy(data_hbm.at[pl.ds(rowof(ub + j), IDROW)],
                                  buf, lsem).wait()
            pltpu.async_copy(buf, acc_sh.at[ibufs.at[up, j]], ssem, add=True)
            if j == 0:
                @pl.when(u > 0)
                def _():
                    pltpu.make_async_copy(
                        dbuf1, acc_sh.at[ibufs.at[1 - up, 7]], ss1).wait()
            else:
                pltpu.make_async_copy(
                    obuf, acc_sh.at[ibufs.at[up, j - 1]], ossem).wait()
            pltpu.async_copy(data_hbm.at[pl.ds(rowof(ub + j + 1), IDROW)],
                             obuf, olsem)
        nxt = pl.multiple_of(
            jnp.minimum((start_unit + u + 1) * 8, IDROWS - 8 - 4), 8)
        pltpu.sync_copy(ids_hbm.at[pl.ds(nxt, 8)], ibufs.at[1 - up])
        return carry

    lax.fori_loop(0, n_units, unit_body, 0)

    up_last = lax.rem(n_units - 1, 2)
    pltpu.make_async_copy(dbuf1, acc_sh.at[ibufs.at[up_last, 7]], ss1).wait()
    pltpu.make_async_copy(
        data_hbm.at[pl.ds(rowof(block0 + n_units * 8), IDROW)],
        dbuf0, sl0).wait()

    @pl.when(w == NW - 1)
    def _tail():
        pltpu.sync_copy(ids_hbm.at[pl.ds(UNITS * 8, 4)],
                        ibufs.at[0, pl.ds(0, 4)])
        for j in range(4):
            row0 = pl.multiple_of((UNITS * 8 + j) * IDROW, 8)
            pltpu.sync_copy(data_hbm.at[pl.ds(row0, IDROW)], dbuf0)
            pltpu.sync_copy(dbuf0, acc_sh.at[ibufs.at[0, j]], add=True)

    plsc.subcore_barrier()

    rtail = SEG_PER_TILE - 4 * IDROW

    def _racc(k):
        rows = IDROW if k < 4 else rtail
        return acc_sh.at[pl.ds(base + k * IDROW, rows)]

    def _rbuf(k):
        buf = bufs[k % 2]
        return buf if k < 4 else buf.at[pl.ds(0, rtail)]

    def _rout(k):
        rows = IDROW if k < 4 else rtail
        return out_hbm.at[c, pl.ds(base + k * IDROW, rows)]

    pltpu.async_copy(_racc(0), _rbuf(0), lsems[0])
    for k in range(5):
        pltpu.make_async_copy(_racc(k), _rbuf(k), lsems[k % 2]).wait()
        if k >= 1:
            pltpu.make_async_copy(_rbuf(k - 1), _rout(k - 1),
                                  ssems[(k - 1) % 2]).wait()
        if k < 4:
            pltpu.async_copy(_racc(k + 1), _rbuf(k + 1), lsems[(k + 1) % 2])
        pltpu.async_copy(_rbuf(k), _rout(k), ssems[k % 2])
    pltpu.make_async_copy(_rbuf(4), _rout(4), ssems[0]).wait()

    @pl.when(s == 0)
    def _write_tail():
        pltpu.sync_copy(acc_sh.at[pl.ds(16 * SEG_PER_TILE, SEG_TAIL)],
                        dbuf0.at[pl.ds(0, SEG_TAIL)])
        pltpu.sync_copy(dbuf0.at[pl.ds(0, SEG_TAIL)],
                        out_hbm.at[c, pl.ds(16 * SEG_PER_TILE, SEG_TAIL)])


def _combine_body(p_ref, o_ref):
    o_ref[...] = p_ref[0] + p_ref[1]


def _combine(partials):
    return pl.pallas_call(
        _combine_body,
        grid=(5,),
        in_specs=[pl.BlockSpec((2, S // 5, D), lambda i: (0, i, 0))],
        out_specs=pl.BlockSpec((S // 5, D), lambda i: (i, 0)),
        out_shape=jax.ShapeDtypeStruct((S, D), jnp.float32),
    )(partials)


def kernel(data, segment_ids):
    ids2d = segment_ids.astype(jnp.int32).reshape(IDROWS, IDROW)
    zeros = jnp.zeros((IDROW, D), jnp.float32)
    partials = _seg_sum_sc(data, ids2d, zeros)
    return _combine(partials)


# speedup vs baseline: 8.8842x; 1.0066x over previous
import functools

import jax
import jax.numpy as jnp
from jax import lax
from jax.experimental import pallas as pl
from jax.experimental.pallas import tpu as pltpu
from jax.experimental.pallas import tpu_sc as plsc

N = 320000
D = 128
S = 10000

IDROW = 128
IDROWS = N // IDROW
IDROWS_PAD = 2504
NW = 32
UNITS = IDROWS // 8
BIGW = 24
SEG_PER_TILE = 624
SEG_TAIL = S - 16 * SEG_PER_TILE

_mesh = plsc.VectorSubcoreMesh(core_axis_name="c", subcore_axis_name="s")


@functools.partial(
    pl.kernel,
    out_type=jax.ShapeDtypeStruct((2, S, D), jnp.float32),
    mesh=_mesh,
    scratch_types=[
        pltpu.VMEM((IDROW, D), jnp.float32),
        pltpu.VMEM((IDROW, D), jnp.float32),
        pltpu.VMEM((2, 8, IDROW), jnp.int32),
        pltpu.VMEM_SHARED((S, D), jnp.float32),
        pltpu.SemaphoreType.DMA,
        pltpu.SemaphoreType.DMA,
        pltpu.SemaphoreType.DMA,
        pltpu.SemaphoreType.DMA,
        pltpu.SemaphoreType.DMA,
    ],
)
def _seg_sum_sc(data_hbm, ids_hbm, zeros_hbm, out_hbm, dbuf0, dbuf1, ibufs,
                acc_sh, sl0, sl1, ss0, ss1, si):
    c = lax.axis_index("c")
    s = lax.axis_index("s")
    w = c * 16 + s
    bufs = (dbuf0, dbuf1)
    lsems = (sl0, sl1)
    ssems = (ss0, ss1)

    start_unit = w * 10 - jnp.maximum(w - BIGW, 0)
    n_units = 10 - (w >= BIGW).astype(jnp.int32)
    block0 = start_unit * 8

    def rowof(b):
        return pl.multiple_of(jnp.minimum(b * IDROW, N - IDROW), 8)

    pltpu.async_copy(data_hbm.at[pl.ds(rowof(block0), IDROW)], dbuf0, sl0)
    pltpu.async_copy(ids_hbm.at[pl.ds(pl.multiple_of(block0, 8), 8)],
                     ibufs.at[0], sl1)

    pltpu.sync_copy(zeros_hbm, dbuf1)
    base = pl.multiple_of(s * SEG_PER_TILE, 8)
    ztail = SEG_PER_TILE - 4 * IDROW
    for k in range(4):
        pltpu.async_copy(dbuf1, acc_sh.at[pl.ds(base + k * IDROW, IDROW)],
                         ss0)
    pltpu.async_copy(dbuf1.at[pl.ds(0, ztail)],
                     acc_sh.at[pl.ds(base + 4 * IDROW, ztail)], ss1)

    @pl.when(s == 0)
    def _zero_tail():
        pltpu.sync_copy(dbuf1.at[pl.ds(0, SEG_TAIL)],
                        acc_sh.at[pl.ds(16 * SEG_PER_TILE, SEG_TAIL)])

    for k in range(4):
        pltpu.make_async_copy(
            dbuf1, acc_sh.at[pl.ds(base + k * IDROW, IDROW)], ss0).wait()
    pltpu.make_async_copy(dbuf1.at[pl.ds(0, ztail)],
                          acc_sh.at[pl.ds(base + 4 * IDROW, ztail)],
                          ss1).wait()
    pltpu.make_async_copy(ids_hbm.at[pl.ds(pl.multiple_of(block0, 8), 8)],
                          ibufs.at[0], sl1).wait()

    plsc.subcore_barrier()

    def nxtrow(u):
        return pl.multiple_of(
            jnp.minimum((start_unit + u + 1) * 8, IDROWS - 8 - 4), 8)

    def unit_body(u, carry):
        up = lax.rem(u, 2)
        ub = block0 + u * 8
        for j in range(8):
            m = j % 2
            buf, lsem, ssem = bufs[m], lsems[m], ssems[m]
            obuf, olsem, ossem = bufs[1 - m], lsems[1 - m], ssems[1 - m]
            pltpu.make_async_copy(data_hbm.at[pl.ds(rowof(ub + j), IDROW)],
                                  buf, lsem).wait()
            pltpu.async_copy(buf, acc_sh.at[ibufs.at[up, j]], ssem, add=True)
            if j == 0:
                @pl.when(u > 0)
                def _():
                    pltpu.make_async_copy(
                        dbuf1, acc_sh.at[ibufs.at[1 - up, 7]], ss1).wait()
                pltpu.async_copy(ids_hbm.at[pl.ds(nxtrow(u), 8)],
                                 ibufs.at[1 - up], si)
            else:
                pltpu.make_async_copy(
                    obuf, acc_sh.at[ibufs.at[up, j - 1]], ossem).wait()
            pltpu.async_copy(data_hbm.at[pl.ds(rowof(ub + j + 1), IDROW)],
                             obuf, olsem)
        pltpu.make_async_copy(ids_hbm.at[pl.ds(nxtrow(u), 8)],
                              ibufs.at[1 - up], si).wait()
        return carry

    lax.fori_loop(0, n_units, unit_body, 0)

    up_last = lax.rem(n_units - 1, 2)
    pltpu.make_async_copy(dbuf1, acc_sh.at[ibufs.at[up_last, 7]], ss1).wait()
    pltpu.make_async_copy(
        data_hbm.at[pl.ds(rowof(block0 + n_units * 8), IDROW)],
        dbuf0, sl0).wait()

    @pl.when(w == NW - 1)
    def _tail():
        pltpu.sync_copy(ids_hbm.at[pl.ds(UNITS * 8, 4)],
                        ibufs.at[0, pl.ds(0, 4)])
        for j in range(4):
            row0 = pl.multiple_of((UNITS * 8 + j) * IDROW, 8)
            pltpu.sync_copy(data_hbm.at[pl.ds(row0, IDROW)], dbuf0)
            pltpu.sync_copy(dbuf0, acc_sh.at[ibufs.at[0, j]], add=True)

    plsc.subcore_barrier()

    rtail = SEG_PER_TILE - 4 * IDROW

    def _racc(k):
        rows = IDROW if k < 4 else rtail
        return acc_sh.at[pl.ds(base + k * IDROW, rows)]

    def _rbuf(k):
        buf = bufs[k % 2]
        return buf if k < 4 else buf.at[pl.ds(0, rtail)]

    def _rout(k):
        rows = IDROW if k < 4 else rtail
        return out_hbm.at[c, pl.ds(base + k * IDROW, rows)]

    pltpu.async_copy(_racc(0), _rbuf(0), lsems[0])
    for k in range(5):
        pltpu.make_async_copy(_racc(k), _rbuf(k), lsems[k % 2]).wait()
        if k >= 1:
            pltpu.make_async_copy(_rbuf(k - 1), _rout(k - 1),
                                  ssems[(k - 1) % 2]).wait()
        if k < 4:
            pltpu.async_copy(_racc(k + 1), _rbuf(k + 1), lsems[(k + 1) % 2])
        pltpu.async_copy(_rbuf(k), _rout(k), ssems[k % 2])
    pltpu.make_async_copy(_rbuf(4), _rout(4), ssems[0]).wait()

    @pl.when(s == 0)
    def _write_tail():
        pltpu.sync_copy(acc_sh.at[pl.ds(16 * SEG_PER_TILE, SEG_TAIL)],
                        dbuf0.at[pl.ds(0, SEG_TAIL)])
        pltpu.sync_copy(dbuf0.at[pl.ds(0, SEG_TAIL)],
                        out_hbm.at[c, pl.ds(16 * SEG_PER_TILE, SEG_TAIL)])


def _combine_body(p_ref, o_ref):
    o_ref[...] = p_ref[0] + p_ref[1]


def _combine(partials):
    return pl.pallas_call(
        _combine_body,
        grid=(5,),
        in_specs=[pl.BlockSpec((2, S // 5, D), lambda i: (0, i, 0))],
        out_specs=pl.BlockSpec((S // 5, D), lambda i: (i, 0)),
        out_shape=jax.ShapeDtypeStruct((S, D), jnp.float32),
    )(partials)


def kernel(data, segment_ids):
    ids2d = segment_ids.astype(jnp.int32).reshape(IDROWS, IDROW)
    zeros = jnp.zeros((IDROW, D), jnp.float32)
    partials = _seg_sum_sc(data, ids2d, zeros)
    return _combine(partials)
